# baseline (speedup 1.0000x reference)
import functools

import jax
import jax.numpy as jnp
from jax.experimental import pallas as pl
from jax.experimental.pallas import tpu as pltpu
from jax.experimental.pallas import tpu_sc as plsc

_NUM_LABELED = 100000
_HIDDEN = 64
_CLASSES = 100
_BATCH = 1024
_LABELED_WEIGHT = 0.7
_EPS = 1e-8

_K_TILE = 8192
_N_STEPS = -(-_NUM_LABELED // _K_TILE)
_LANE = 128
_N_SLICES = _K_TILE // _LANE
_PAYLOAD_MASK = 0x3FF

_CPAD = 128
_GATHER_WINDOW = 128


def _simarg_body(wd_ref, mt_ref, lot_ref, idx_ref, po_ref, acc_ref):
    step = pl.program_id(0)
    po_ref[...] = jnp.concatenate(
        [
            lot_ref[...].T,
            jnp.zeros((_K_TILE, _CPAD - _CLASSES), jnp.float32),
        ],
        axis=1,
    )

    cols = (
        jax.lax.broadcasted_iota(jnp.int32, (1, _K_TILE), 1) + step * _K_TILE
    )
    valid = cols < _NUM_LABELED
    mt = jnp.where(valid, mt_ref[...], 0.0)
    ss = jnp.sum(mt * mt, axis=0, keepdims=True)
    inv = jax.lax.rsqrt(jnp.maximum(ss, _EPS * _EPS))
    aug = jnp.where(valid, 2.0, -1e30).astype(jnp.float32)
    mb = jnp.concatenate([mt * inv, aug], axis=0).astype(jnp.bfloat16)

    s = jax.lax.dot_general(
        wd_ref[...], mb, (((1,), (0,)), ((), ())),
        preferred_element_type=jnp.float32,
    )
    bits = jax.lax.bitcast_convert_type(s, jnp.int32)

    @pl.when(step == 0)
    def _():
        acc_ref[...] = jnp.zeros((_BATCH, _LANE), jnp.float32)

    acc = acc_ref[...]
    for j in range(_N_SLICES):
        kj = (bits[:, j * _LANE:(j + 1) * _LANE] & jnp.int32(~_PAYLOAD_MASK)) | (
            step * _N_SLICES + j
        )
        acc = jnp.maximum(acc, jax.lax.bitcast_convert_type(kj, jnp.float32))
    acc_ref[...] = acc

    @pl.when(step == _N_STEPS - 1)
    def _():
        best = jnp.max(acc, axis=1, keepdims=True)
        lanes = jax.lax.broadcasted_iota(jnp.int32, (_BATCH, _LANE), 1)
        lane = jnp.min(
            jnp.where(acc == best, lanes, jnp.int32(_LANE)),
            axis=1,
            keepdims=True,
        )
        bi = jax.lax.bitcast_convert_type(best, jnp.int32)
        idx_ref[...] = (bi & _PAYLOAD_MASK) * _LANE + lane


def _simarg(weak_data_aug, labeled_memory_t, labeled_logits_t):
    return pl.pallas_call(
        _simarg_body,
        grid=(_N_STEPS,),
        in_specs=[
            pl.BlockSpec((_BATCH, _HIDDEN + 1), lambda i: (0, 0)),
            pl.BlockSpec((_HIDDEN, _K_TILE), lambda i: (0, i)),
            pl.BlockSpec((_CLASSES, _K_TILE), lambda i: (0, i)),
        ],
        out_specs=[
            pl.BlockSpec((_BATCH, 1), lambda i: (0, 0)),
            pl.BlockSpec((_K_TILE, _CPAD), lambda i: (i, 0)),
        ],
        out_shape=[
            jax.ShapeDtypeStruct((_BATCH, 1), jnp.int32),
            jax.ShapeDtypeStruct((_NUM_LABELED, _CPAD), jnp.float32),
        ],
        scratch_shapes=[
            pltpu.VMEM((_BATCH, _LANE), jnp.float32),
        ],
    )(weak_data_aug, labeled_memory_t, labeled_logits_t)


def _gather_sc(ids_2d, table):
    @functools.partial(
        pl.kernel,
        out_type=jax.ShapeDtypeStruct((_BATCH, _CPAD), jnp.float32),
        mesh=plsc.VectorSubcoreMesh(
            core_axis_name="core", subcore_axis_name="subcore"
        ),
    )
    def k(i_hbm, t_hbm, o_hbm):
        def body(i_vmem, o_vmem):
            pltpu.sync_copy(t_hbm.at[i_vmem.at[0]], o_vmem)

        pltpu.emit_pipeline(
            body,
            grid=(_BATCH // _GATHER_WINDOW,),
            in_specs=[
                pl.BlockSpec((1, _GATHER_WINDOW), index_map=lambda i: (0, i))
            ],
            out_specs=[
                pl.BlockSpec(
                    (_GATHER_WINDOW, _CPAD), index_map=lambda i: (i, 0)
                )
            ],
            core_axis_name=("core", "subcore"),
            dimension_semantics=(pltpu.PARALLEL,),
        )(i_hbm, o_hbm)

    return k(ids_2d, table)


def _blend_body(g_ref, w_ref, o_ref):
    o_ref[...] = g_ref[:, : _CLASSES] * _LABELED_WEIGHT + (
        1.0 - _NUM_LABELED
    ) * w_ref[...]


def _blend(g, weak_logits):
    return pl.pallas_call(
        _blend_body,
        out_shape=jax.ShapeDtypeStruct((_BATCH, _CLASSES), jnp.float32),
    )(g, weak_logits)


def kernel(weak_data, weak_logits, labeled_memory, labeled_logits):
    wd_aug = jnp.concatenate(
        [weak_data, jnp.ones((_BATCH, 1), jnp.float32)], axis=1
    ).astype(jnp.bfloat16)
    ids, table = _simarg(wd_aug, labeled_memory.T, labeled_logits.T)
    ids_2d = ids.reshape(1, _BATCH)
    g = _gather_sc(ids_2d, table)
    return _blend(g, weak_logits)


# speedup vs baseline: 1.0134x; 1.0134x over previous
import functools

import jax
import jax.numpy as jnp
from jax.experimental import pallas as pl
from jax.experimental.pallas import tpu as pltpu
from jax.experimental.pallas import tpu_sc as plsc

_NUM_LABELED = 100000
_HIDDEN = 64
_CLASSES = 100
_BATCH = 1024
_LABELED_WEIGHT = 0.7
_EPS = 1e-8

_K_TILE = 4096
_N_STEPS = -(-_NUM_LABELED // _K_TILE)
_LANE = 128
_N_SLICES = _K_TILE // _LANE
_PAYLOAD_MASK = 0x3FF

_CPAD = 128
_GATHER_WINDOW = 128


def _simarg_body(wd_ref, mt_ref, lot_ref, idx_ref, po_ref, acc_ref):
    step = pl.program_id(0)
    po_ref[...] = jnp.concatenate(
        [
            lot_ref[...].T,
            jnp.zeros((_K_TILE, _CPAD - _CLASSES), jnp.float32),
        ],
        axis=1,
    )

    cols = (
        jax.lax.broadcasted_iota(jnp.int32, (1, _K_TILE), 1) + step * _K_TILE
    )
    valid = cols < _NUM_LABELED
    mt = jnp.where(valid, mt_ref[...], 0.0)
    ss = jnp.sum(mt * mt, axis=0, keepdims=True)
    inv = jax.lax.rsqrt(jnp.maximum(ss, _EPS * _EPS))
    aug = jnp.where(valid, 2.0, -1e30).astype(jnp.float32)
    mb = jnp.concatenate([mt * inv, aug], axis=0).astype(jnp.bfloat16)

    s = jax.lax.dot_general(
        wd_ref[...], mb, (((1,), (0,)), ((), ())),
        preferred_element_type=jnp.float32,
    )
    bits = jax.lax.bitcast_convert_type(s, jnp.int32)

    @pl.when(step == 0)
    def _():
        acc_ref[...] = jnp.zeros((_BATCH, _LANE), jnp.float32)

    acc = acc_ref[...]
    for j in range(_N_SLICES):
        kj = (bits[:, j * _LANE:(j + 1) * _LANE] & jnp.int32(~_PAYLOAD_MASK)) | (
            step * _N_SLICES + j
        )
        acc = jnp.maximum(acc, jax.lax.bitcast_convert_type(kj, jnp.float32))
    acc_ref[...] = acc

    @pl.when(step == _N_STEPS - 1)
    def _():
        best = jnp.max(acc, axis=1, keepdims=True)
        lanes = jax.lax.broadcasted_iota(jnp.int32, (_BATCH, _LANE), 1)
        lane = jnp.min(
            jnp.where(acc == best, lanes, jnp.int32(_LANE)),
            axis=1,
            keepdims=True,
        )
        bi = jax.lax.bitcast_convert_type(best, jnp.int32)
        idx_ref[...] = (bi & _PAYLOAD_MASK) * _LANE + lane


def _simarg(weak_data_aug, labeled_memory_t, labeled_logits_t):
    return pl.pallas_call(
        _simarg_body,
        grid=(_N_STEPS,),
        in_specs=[
            pl.BlockSpec((_BATCH, _HIDDEN + 1), lambda i: (0, 0)),
            pl.BlockSpec((_HIDDEN, _K_TILE), lambda i: (0, i)),
            pl.BlockSpec((_CLASSES, _K_TILE), lambda i: (0, i)),
        ],
        out_specs=[
            pl.BlockSpec((_BATCH, 1), lambda i: (0, 0)),
            pl.BlockSpec((_K_TILE, _CPAD), lambda i: (i, 0)),
        ],
        out_shape=[
            jax.ShapeDtypeStruct((_BATCH, 1), jnp.int32),
            jax.ShapeDtypeStruct((_NUM_LABELED, _CPAD), jnp.float32),
        ],
        scratch_shapes=[
            pltpu.VMEM((_BATCH, _LANE), jnp.float32),
        ],
    )(weak_data_aug, labeled_memory_t, labeled_logits_t)


def _gather_sc(ids_2d, table):
    @functools.partial(
        pl.kernel,
        out_type=jax.ShapeDtypeStruct((_BATCH, _CPAD), jnp.float32),
        mesh=plsc.VectorSubcoreMesh(
            core_axis_name="core", subcore_axis_name="subcore"
        ),
    )
    def k(i_hbm, t_hbm, o_hbm):
        def body(i_vmem, o_vmem):
            pltpu.sync_copy(t_hbm.at[i_vmem.at[0]], o_vmem)

        pltpu.emit_pipeline(
            body,
            grid=(_BATCH // _GATHER_WINDOW,),
            in_specs=[
                pl.BlockSpec((1, _GATHER_WINDOW), index_map=lambda i: (0, i))
            ],
            out_specs=[
                pl.BlockSpec(
                    (_GATHER_WINDOW, _CPAD), index_map=lambda i: (i, 0)
                )
            ],
            core_axis_name=("core", "subcore"),
            dimension_semantics=(pltpu.PARALLEL,),
        )(i_hbm, o_hbm)

    return k(ids_2d, table)


def _blend_body(g_ref, w_ref, o_ref):
    o_ref[...] = g_ref[:, : _CLASSES] * _LABELED_WEIGHT + (
        1.0 - _NUM_LABELED
    ) * w_ref[...]


def _blend(g, weak_logits):
    return pl.pallas_call(
        _blend_body,
        out_shape=jax.ShapeDtypeStruct((_BATCH, _CLASSES), jnp.float32),
    )(g, weak_logits)


def kernel(weak_data, weak_logits, labeled_memory, labeled_logits):
    wd_aug = jnp.concatenate(
        [weak_data, jnp.ones((_BATCH, 1), jnp.float32)], axis=1
    ).astype(jnp.bfloat16)
    ids, table = _simarg(wd_aug, labeled_memory.T, labeled_logits.T)
    ids_2d = ids.reshape(1, _BATCH)
    g = _gather_sc(ids_2d, table)
    return _blend(g, weak_logits)
